# Initial kernel scaffold; baseline (speedup 1.0000x reference)
import jax
import jax.numpy as jnp
from jax.experimental import pallas as pl


def kernel(embed, edge_index, W_lin, b_lin, W_gat, attn_l, attn_r, b_gat):
    raise NotImplementedError("write your pallas kernel here")


# speedup vs baseline: 1.1845x; 1.1845x over previous
import jax
import jax.numpy as jnp
from jax.experimental import pallas as pl

_N = 50000
_E = 800000
_BN = 2000
_ER = 200
_EC = 8000
_BE = 6400


def _dense_body(embed_ref, wl_ref, bl_ref, wg_ref, al_ref, ar_ref,
                h_ref, el_ref, er_ref):
    x = jnp.maximum(embed_ref[...] @ wl_ref[...] + bl_ref[...], 0.0)
    h = x @ wg_ref[...]
    h_ref[...] = h
    el_ref[...] = h @ al_ref[...]
    er_ref[...] = h @ ar_ref[...]


def _edge_body(elg_ref, erg_ref, ee_ref):
    s = elg_ref[...] + erg_ref[...]
    e = jnp.where(s >= 0, s, 0.2 * s)
    ee_ref[...] = jnp.exp(e)


def _msg_body(h_ref, a_ref, d_ref, m_ref):
    alpha = a_ref[...] / (d_ref[...] + 1e-12)
    m_ref[...] = h_ref[...] * alpha


def _out_body(agg_ref, bg_ref, mu_ref, lv_ref):
    y = agg_ref[...] + bg_ref[...]
    mu_ref[...] = y[:, :64]
    lv_ref[...] = jnp.tanh(y[:, 64:])


def kernel(embed, edge_index, W_lin, b_lin, W_gat, attn_l, attn_r, b_gat):
    h, el2, er2 = pl.pallas_call(
        _dense_body,
        grid=(_N // _BN,),
        in_specs=[
            pl.BlockSpec((_BN, 384), lambda i: (i, 0)),
            pl.BlockSpec((384, 64), lambda i: (0, 0)),
            pl.BlockSpec((1, 64), lambda i: (0, 0)),
            pl.BlockSpec((64, 128), lambda i: (0, 0)),
            pl.BlockSpec((128, 1), lambda i: (0, 0)),
            pl.BlockSpec((128, 1), lambda i: (0, 0)),
        ],
        out_specs=[
            pl.BlockSpec((_BN, 128), lambda i: (i, 0)),
            pl.BlockSpec((_BN, 1), lambda i: (i, 0)),
            pl.BlockSpec((_BN, 1), lambda i: (i, 0)),
        ],
        out_shape=[
            jax.ShapeDtypeStruct((_N, 128), jnp.float32),
            jax.ShapeDtypeStruct((_N, 1), jnp.float32),
            jax.ShapeDtypeStruct((_N, 1), jnp.float32),
        ],
    )(embed, W_lin, b_lin.reshape(1, 64), W_gat,
      attn_l.reshape(128, 1), attn_r.reshape(128, 1))

    src2 = jnp.concatenate([edge_index[0], edge_index[1]])
    dst2 = jnp.concatenate([edge_index[1], edge_index[0]])
    el = el2[:, 0]
    er = er2[:, 0]

    elg = el[src2].reshape(_ER, _EC)
    erg = er[dst2].reshape(_ER, _EC)
    ee = pl.pallas_call(
        _edge_body,
        grid=(_ER // 8,),
        in_specs=[pl.BlockSpec((8, _EC), lambda i: (i, 0))] * 2,
        out_specs=pl.BlockSpec((8, _EC), lambda i: (i, 0)),
        out_shape=jax.ShapeDtypeStruct((_ER, _EC), jnp.float32),
    )(elg, erg).reshape(-1)

    denom = jax.ops.segment_sum(ee, dst2, num_segments=_N)
    hsrc = h[src2]
    msg = pl.pallas_call(
        _msg_body,
        grid=(2 * _E // _BE,),
        in_specs=[
            pl.BlockSpec((_BE, 128), lambda i: (i, 0)),
            pl.BlockSpec((_BE, 1), lambda i: (i, 0)),
            pl.BlockSpec((_BE, 1), lambda i: (i, 0)),
        ],
        out_specs=pl.BlockSpec((_BE, 128), lambda i: (i, 0)),
        out_shape=jax.ShapeDtypeStruct((2 * _E, 128), jnp.float32),
    )(hsrc, ee.reshape(-1, 1), denom[dst2].reshape(-1, 1))

    agg = jax.ops.segment_sum(msg, dst2, num_segments=_N)

    mu, lv = pl.pallas_call(
        _out_body,
        grid=(_N // _BN,),
        in_specs=[
            pl.BlockSpec((_BN, 128), lambda i: (i, 0)),
            pl.BlockSpec((1, 128), lambda i: (0, 0)),
        ],
        out_specs=[pl.BlockSpec((_BN, 64), lambda i: (i, 0))] * 2,
        out_shape=[jax.ShapeDtypeStruct((_N, 64), jnp.float32)] * 2,
    )(agg, b_gat.reshape(1, 128))
    return mu, lv
